# baseline (speedup 1.0000x reference)
import functools

import jax
import jax.numpy as jnp
from jax import lax
from jax.experimental import pallas as pl
from jax.experimental.pallas import tpu as pltpu
from jax.experimental.pallas import tpu_sc as plsc

_N = 10000
_D = 128
_E = 320000
_K = 10
_ALPHA = 0.1

_NC = 2
_NS = 16
_DH = _D // _NC
_CH = 128
_KB = 2
_JPW = 158
_EPAD = _NS * _JPW * _CH
_R = 10240
_ZC = _R // (_NS * _CH)

_mesh = plsc.VectorSubcoreMesh(core_axis_name="c", subcore_axis_name="s")


def _agg_body(h_hbm, src_hbm, dst_hbm, out_hbm,
              src_v, dst_v, rows_v, zeros_v, wb_v, acc_sh, sem):
    c = lax.axis_index("c")
    s = lax.axis_index("s")
    base = s * (_ZC * _CH)

    def zfill(i, _):
        for l in range(_DH // 16):
            zeros_v[i, pl.ds(l * 16, 16)] = jnp.zeros((16,), jnp.float32)
        return 0
    lax.fori_loop(0, _CH, zfill, 0)

    def zchunk(i, _):
        pltpu.sync_copy(zeros_v, acc_sh.at[pl.ds(base + i * _CH, _CH)])
        return 0
    lax.fori_loop(0, _ZC, zchunk, 0)

    pltpu.sync_copy(src_hbm.at[s], src_v)
    pltpu.sync_copy(dst_hbm.at[s], dst_v)
    plsc.subcore_barrier()

    def group(g, _):
        j0 = g * _KB

        def fire(i, _):
            pltpu.async_copy(h_hbm.at[c].at[src_v.at[j0 + i]],
                             rows_v.at[pl.ds(i * _CH, _CH)], sem)
            return 0
        lax.fori_loop(0, _KB, fire, 0)

        def drain(i, _):
            pltpu.make_async_copy(h_hbm.at[c].at[src_v.at[j0 + i]],
                                  rows_v.at[pl.ds(i * _CH, _CH)], sem).wait()
            return 0
        lax.fori_loop(0, _KB, drain, 0)

        def scat(i, _):
            pltpu.sync_copy(rows_v.at[pl.ds(i * _CH, _CH)],
                            acc_sh.at[dst_v.at[j0 + i]], add=True)
            return 0
        lax.fori_loop(0, _KB, scat, 0)
        return 0
    lax.fori_loop(0, _JPW // _KB, group, 0)
    plsc.subcore_barrier()

    def wchunk(i, _):
        pltpu.sync_copy(acc_sh.at[pl.ds(base + i * _CH, _CH)], wb_v)
        pltpu.sync_copy(wb_v, out_hbm.at[c, pl.ds(base + i * _CH, _CH)])
        return 0
    lax.fori_loop(0, _ZC, wchunk, 0)


_agg_call = functools.partial(
    pl.kernel,
    out_type=jax.ShapeDtypeStruct((_NC, _R, _DH), jnp.float32),
    mesh=_mesh,
    compiler_params=pltpu.CompilerParams(use_tc_tiling_on_sc=False),
    scratch_types=[
        pltpu.VMEM((_JPW, _CH), jnp.int32),
        pltpu.VMEM((_JPW, _CH), jnp.int32),
        pltpu.VMEM((_KB * _CH, _DH), jnp.float32),
        pltpu.VMEM((_CH, _DH), jnp.float32),
        pltpu.VMEM((_CH, _DH), jnp.float32),
        pltpu.VMEM_SHARED((_R, _DH), jnp.float32),
        pltpu.SemaphoreType.DMA,
    ],
)(_agg_body)


_BLK = 512


def _combine_body(a, av, bv, o):
    o[...] = a[...] * av[...] + bv[...]


def _combine_call(a, av, bv):
    spec = pl.BlockSpec((_NC, _BLK, _DH), lambda i: (0, i, 0))
    return pl.pallas_call(
        _combine_body,
        grid=(_R // _BLK,),
        in_specs=[spec] * 3,
        out_specs=spec,
        out_shape=jax.ShapeDtypeStruct((_NC, _R, _DH), jnp.float32),
    )(a, av, bv)


def _mlp_body(hv, rv, w0, b0, w1, b1, o):
    g = hv[...] * rv[...]
    h = jnp.concatenate([g[0], g[1]], axis=-1)
    z = jnp.maximum(
        jnp.dot(h, w0[...], preferred_element_type=jnp.float32) + b0[...], 0.0)
    o[...] = jnp.dot(z, w1[...], preferred_element_type=jnp.float32) + b1[...]


def _mlp_call(hv, rv, w0, b0, w1, b1):
    H = w0.shape[1]
    P = w1.shape[1]
    return pl.pallas_call(
        _mlp_body,
        grid=(_R // _BLK,),
        in_specs=[
            pl.BlockSpec((_NC, _BLK, _DH), lambda i: (0, i, 0)),
            pl.BlockSpec((_NC, _BLK, _DH), lambda i: (0, i, 0)),
            pl.BlockSpec((_D, H), lambda i: (0, 0)),
            pl.BlockSpec((1, H), lambda i: (0, 0)),
            pl.BlockSpec((H, P), lambda i: (0, 0)),
            pl.BlockSpec((1, P), lambda i: (0, 0)),
        ],
        out_specs=pl.BlockSpec((_BLK, P), lambda i: (i, 0)),
        out_shape=jax.ShapeDtypeStruct((_R, P), jnp.float32),
    )(hv, rv, w0, b0, w1, b1)


def _to_split(m):
    return m.reshape(_R, _NC, _DH).transpose(1, 0, 2)


def kernel(x, edge_index, W0, b0, W1, b1):
    src = edge_index[0].astype(jnp.int32)
    dst = edge_index[1].astype(jnp.int32)
    pad = jnp.full((_EPAD - _E,), _N, jnp.int32)
    src3 = jnp.concatenate([src, pad]).reshape(_NS, _JPW, _CH)
    dst3 = jnp.concatenate([dst, pad]).reshape(_NS, _JPW, _CH)

    ones_h = jnp.zeros((_NC, _R, _DH), jnp.float32).at[:, :_N].set(1.0)
    dparts = _agg_call(ones_h, src3, dst3)
    deg_in = dparts[0, :_N, 0]
    ones_h2 = ones_h + 0.0 * dparts
    dparts = _agg_call(ones_h2, dst3, src3)
    deg_out = dparts[0, :_N, 0]
    in_norm = lax.rsqrt(jnp.maximum(deg_in, 1.0))
    out_norm = lax.rsqrt(jnp.maximum(deg_out, 1.0))

    zpad = jnp.zeros((_R - _N,), jnp.float32)
    opad = jnp.ones((_R - _N,), jnp.float32)
    onp = jnp.concatenate([out_norm, opad])
    inp_ = jnp.concatenate([in_norm, zpad])
    xpad = jnp.pad(x, ((0, _R - _N), (0, 0)))

    scale = (1.0 - _ALPHA)
    av = _to_split(jnp.broadcast_to((scale * onp * inp_)[:, None], (_R, _D)))
    bv = _to_split((_ALPHA * onp)[:, None] * xpad)
    recip = jnp.concatenate([jnp.sqrt(jnp.maximum(deg_out, 1.0)), opad])
    rv = _to_split(jnp.broadcast_to(recip[:, None], (_R, _D)))

    def step(_, h):
        return _combine_call(_agg_call(h, src3, dst3), av, bv)

    h = lax.fori_loop(0, _K, step, _to_split(onp[:, None] * xpad))

    H = W0.shape[1]
    C = W1.shape[1]
    P = 128
    W1p = jnp.pad(W1, ((0, 0), (0, P - C)))
    b1p = jnp.pad(b1, (0, P - C)).reshape(1, P)
    b0r = b0.reshape(1, H)
    logits = _mlp_call(h, rv, W0, b0r, W1p, b1p)
    return logits[:_N, :C]


# speedup vs baseline: 1.0890x; 1.0890x over previous
import functools

import jax
import jax.numpy as jnp
from jax import lax
from jax.experimental import pallas as pl
from jax.experimental.pallas import tpu as pltpu
from jax.experimental.pallas import tpu_sc as plsc

_N = 10000
_D = 128
_E = 320000
_K = 10
_ALPHA = 0.1

_NC = 2
_NS = 16
_DH = _D // _NC
_CH = 128
_KB = 2
_JPW = 158
_EPAD = _NS * _JPW * _CH
_R = 10240
_ZC = _R // (_NS * _CH)

_mesh = plsc.VectorSubcoreMesh(core_axis_name="c", subcore_axis_name="s")


def _agg_body(h_hbm, src_hbm, dst_hbm, out_hbm,
              src_v, dst_v, rows_v, zeros_v, wb_v, acc_sh, sem):
    c = lax.axis_index("c")
    s = lax.axis_index("s")
    base = s * (_ZC * _CH)

    def zfill(i, _):
        for l in range(_DH // 16):
            zeros_v[i, pl.ds(l * 16, 16)] = jnp.zeros((16,), jnp.float32)
        return 0
    lax.fori_loop(0, _CH, zfill, 0)

    def zchunk(i, _):
        pltpu.sync_copy(zeros_v, acc_sh.at[pl.ds(base + i * _CH, _CH)])
        return 0
    lax.fori_loop(0, _ZC, zchunk, 0)

    pltpu.sync_copy(src_hbm.at[s], src_v)
    pltpu.sync_copy(dst_hbm.at[s], dst_v)
    plsc.subcore_barrier()

    def step(j, _):
        pltpu.async_copy(h_hbm.at[c].at[src_v.at[j]], rows_v, sem).wait()
        pltpu.sync_copy(rows_v, acc_sh.at[dst_v.at[j]], add=True)
        return 0
    lax.fori_loop(0, _JPW, step, 0)
    plsc.subcore_barrier()

    def wchunk(i, _):
        pltpu.sync_copy(acc_sh.at[pl.ds(base + i * _CH, _CH)], wb_v)
        pltpu.sync_copy(wb_v, out_hbm.at[c, pl.ds(base + i * _CH, _CH)])
        return 0
    lax.fori_loop(0, _ZC, wchunk, 0)


_agg_call = functools.partial(
    pl.kernel,
    out_type=jax.ShapeDtypeStruct((_NC, _R, _DH), jnp.float32),
    mesh=_mesh,
    compiler_params=pltpu.CompilerParams(use_tc_tiling_on_sc=False),
    scratch_types=[
        pltpu.VMEM((_JPW, _CH), jnp.int32),
        pltpu.VMEM((_JPW, _CH), jnp.int32),
        pltpu.VMEM((_CH, _DH), jnp.float32),
        pltpu.VMEM((_CH, _DH), jnp.float32),
        pltpu.VMEM((_CH, _DH), jnp.float32),
        pltpu.VMEM_SHARED((_R, _DH), jnp.float32),
        pltpu.SemaphoreType.DMA,
    ],
)(_agg_body)


_DW = 16


def _cnt_body(src_hbm, dst_hbm, out_hbm,
              src_v, dst_v, ones_v, zeros_v, wb_v, deg_sh):
    c = lax.axis_index("c")
    s = lax.axis_index("s")
    base = s * (_ZC * _CH)

    def fill(i, _):
        ones_v[i, pl.ds(0, _DW)] = jnp.full((_DW,), 1.0, jnp.float32)
        zeros_v[i, pl.ds(0, _DW)] = jnp.zeros((_DW,), jnp.float32)
        return 0
    lax.fori_loop(0, _CH, fill, 0)

    def zchunk(i, _):
        pltpu.sync_copy(zeros_v, deg_sh.at[pl.ds(base + i * _CH, _CH)])
        return 0
    lax.fori_loop(0, _ZC, zchunk, 0)

    pltpu.sync_copy(src_hbm.at[s], src_v)
    pltpu.sync_copy(dst_hbm.at[s], dst_v)
    plsc.subcore_barrier()

    @pl.when(c == 0)
    def _():
        def stepd(j, _):
            pltpu.sync_copy(ones_v, deg_sh.at[dst_v.at[j]], add=True)
            return 0
        lax.fori_loop(0, _JPW, stepd, 0)

    @pl.when(c == 1)
    def _():
        def steps_(j, _):
            pltpu.sync_copy(ones_v, deg_sh.at[src_v.at[j]], add=True)
            return 0
        lax.fori_loop(0, _JPW, steps_, 0)
    plsc.subcore_barrier()

    def wchunk(i, _):
        pltpu.sync_copy(deg_sh.at[pl.ds(base + i * _CH, _CH)], wb_v)
        pltpu.sync_copy(wb_v, out_hbm.at[c, pl.ds(base + i * _CH, _CH)])
        return 0
    lax.fori_loop(0, _ZC, wchunk, 0)


_cnt_call = functools.partial(
    pl.kernel,
    out_type=jax.ShapeDtypeStruct((_NC, _R, _DW), jnp.float32),
    mesh=_mesh,
    compiler_params=pltpu.CompilerParams(use_tc_tiling_on_sc=False),
    scratch_types=[
        pltpu.VMEM((_JPW, _CH), jnp.int32),
        pltpu.VMEM((_JPW, _CH), jnp.int32),
        pltpu.VMEM((_CH, _DW), jnp.float32),
        pltpu.VMEM((_CH, _DW), jnp.float32),
        pltpu.VMEM((_CH, _DW), jnp.float32),
        pltpu.VMEM_SHARED((_R, _DW), jnp.float32),
    ],
)(_cnt_body)


_BLK = 512


def _combine_body(a, av, bv, o):
    o[...] = a[...] * av[...] + bv[...]


def _combine_call(a, av, bv):
    spec = pl.BlockSpec((_NC, _BLK, _DH), lambda i: (0, i, 0))
    return pl.pallas_call(
        _combine_body,
        grid=(_R // _BLK,),
        in_specs=[spec] * 3,
        out_specs=spec,
        out_shape=jax.ShapeDtypeStruct((_NC, _R, _DH), jnp.float32),
    )(a, av, bv)


def _mlp_body(hv, rv, w0, b0, w1, b1, o):
    g = hv[...] * rv[...]
    h = jnp.concatenate([g[0], g[1]], axis=-1)
    z = jnp.maximum(
        jnp.dot(h, w0[...], preferred_element_type=jnp.float32) + b0[...], 0.0)
    o[...] = jnp.dot(z, w1[...], preferred_element_type=jnp.float32) + b1[...]


def _mlp_call(hv, rv, w0, b0, w1, b1):
    H = w0.shape[1]
    P = w1.shape[1]
    return pl.pallas_call(
        _mlp_body,
        grid=(_R // _BLK,),
        in_specs=[
            pl.BlockSpec((_NC, _BLK, _DH), lambda i: (0, i, 0)),
            pl.BlockSpec((_NC, _BLK, _DH), lambda i: (0, i, 0)),
            pl.BlockSpec((_D, H), lambda i: (0, 0)),
            pl.BlockSpec((1, H), lambda i: (0, 0)),
            pl.BlockSpec((H, P), lambda i: (0, 0)),
            pl.BlockSpec((1, P), lambda i: (0, 0)),
        ],
        out_specs=pl.BlockSpec((_BLK, P), lambda i: (i, 0)),
        out_shape=jax.ShapeDtypeStruct((_R, P), jnp.float32),
    )(hv, rv, w0, b0, w1, b1)


def _to_split(m):
    return m.reshape(_R, _NC, _DH).transpose(1, 0, 2)


def kernel(x, edge_index, W0, b0, W1, b1):
    src = edge_index[0].astype(jnp.int32)
    dst = edge_index[1].astype(jnp.int32)
    pad = jnp.full((_EPAD - _E,), _N, jnp.int32)
    src3 = jnp.concatenate([src, pad]).reshape(_NS, _JPW, _CH)
    dst3 = jnp.concatenate([dst, pad]).reshape(_NS, _JPW, _CH)

    dcnt = _cnt_call(src3, dst3)
    deg_in = dcnt[0, :_N, 0]
    deg_out = dcnt[1, :_N, 0]
    in_norm = lax.rsqrt(jnp.maximum(deg_in, 1.0))
    out_norm = lax.rsqrt(jnp.maximum(deg_out, 1.0))

    zpad = jnp.zeros((_R - _N,), jnp.float32)
    opad = jnp.ones((_R - _N,), jnp.float32)
    onp = jnp.concatenate([out_norm, opad])
    inp_ = jnp.concatenate([in_norm, zpad])
    xpad = jnp.pad(x, ((0, _R - _N), (0, 0)))

    scale = (1.0 - _ALPHA)
    av = _to_split(jnp.broadcast_to((scale * onp * inp_)[:, None], (_R, _D)))
    bv = _to_split((_ALPHA * onp)[:, None] * xpad)
    recip = jnp.concatenate([jnp.sqrt(jnp.maximum(deg_out, 1.0)), opad])
    rv = _to_split(jnp.broadcast_to(recip[:, None], (_R, _D)))

    def step(_, h):
        return _combine_call(_agg_call(h, src3, dst3), av, bv)

    h = lax.fori_loop(0, _K, step, _to_split(onp[:, None] * xpad))

    H = W0.shape[1]
    C = W1.shape[1]
    P = 128
    W1p = jnp.pad(W1, ((0, 0), (0, P - C)))
    b1p = jnp.pad(b1, (0, P - C)).reshape(1, P)
    b0r = b0.reshape(1, H)
    logits = _mlp_call(h, rv, W0, b0r, W1p, b1p)
    return logits[:_N, :C]


# speedup vs baseline: 1.4954x; 1.3731x over previous
import functools

import jax
import jax.numpy as jnp
from jax import lax
from jax.experimental import pallas as pl
from jax.experimental.pallas import tpu as pltpu
from jax.experimental.pallas import tpu_sc as plsc

_N = 10000
_D = 128
_E = 320000
_K = 10
_ALPHA = 0.1

_NC = 2
_NS = 16
_DH = _D // _NC
_CH = 128
_KB = 2
_JPW = 158
_EPAD = _NS * _JPW * _CH
_R = 10240
_ZC = _R // (_NS * _CH)

_mesh = plsc.VectorSubcoreMesh(core_axis_name="c", subcore_axis_name="s")


def _agg_body(h_hbm, src_hbm, dst_hbm, out_hbm,
              src_v, dst_v, rows0_v, rows1_v, zeros_v, wb_v, acc_sh,
              sem0, sem1):
    c = lax.axis_index("c")
    s = lax.axis_index("s")
    base = s * (_ZC * _CH)

    def zfill(i, _):
        for l in range(_DH // 16):
            zeros_v[i, pl.ds(l * 16, 16)] = jnp.zeros((16,), jnp.float32)
        return 0
    lax.fori_loop(0, _CH, zfill, 0)

    def zchunk(i, _):
        pltpu.sync_copy(zeros_v, acc_sh.at[pl.ds(base + i * _CH, _CH)])
        return 0
    lax.fori_loop(0, _ZC, zchunk, 0)

    pltpu.sync_copy(src_hbm.at[s], src_v)
    pltpu.sync_copy(dst_hbm.at[s], dst_v)
    plsc.subcore_barrier()

    rows = (rows0_v, rows1_v)
    sems = (sem0, sem1)
    pltpu.async_copy(h_hbm.at[c].at[src_v.at[0]], rows0_v, sem0)
    pltpu.async_copy(h_hbm.at[c].at[src_v.at[1]], rows1_v, sem1)

    def step(jj, _):
        for b in range(2):
            j = jj * 2 + b
            pltpu.make_async_copy(h_hbm.at[c].at[src_v.at[j]],
                                  rows[b], sems[b]).wait()
            pltpu.sync_copy(rows[b], acc_sh.at[dst_v.at[j]], add=True)

            @pl.when(j + 2 < _JPW)
            def _():
                pltpu.async_copy(h_hbm.at[c].at[src_v.at[j + 2]],
                                 rows[b], sems[b])
        return 0
    lax.fori_loop(0, _JPW // 2, step, 0)
    plsc.subcore_barrier()

    def wchunk(i, _):
        pltpu.sync_copy(acc_sh.at[pl.ds(base + i * _CH, _CH)], wb_v)
        pltpu.sync_copy(wb_v, out_hbm.at[c, pl.ds(base + i * _CH, _CH)])
        return 0
    lax.fori_loop(0, _ZC, wchunk, 0)


_agg_call = functools.partial(
    pl.kernel,
    out_type=jax.ShapeDtypeStruct((_NC, _R, _DH), jnp.float32),
    mesh=_mesh,
    compiler_params=pltpu.CompilerParams(use_tc_tiling_on_sc=False),
    scratch_types=[
        pltpu.VMEM((_JPW, _CH), jnp.int32),
        pltpu.VMEM((_JPW, _CH), jnp.int32),
        pltpu.VMEM((_CH, _DH), jnp.float32),
        pltpu.VMEM((_CH, _DH), jnp.float32),
        pltpu.VMEM((_CH, _DH), jnp.float32),
        pltpu.VMEM((_CH, _DH), jnp.float32),
        pltpu.VMEM_SHARED((_R, _DH), jnp.float32),
        pltpu.SemaphoreType.DMA,
        pltpu.SemaphoreType.DMA,
    ],
)(_agg_body)


_DW = 16


def _cnt_body(src_hbm, dst_hbm, out_hbm,
              src_v, dst_v, ones_v, zeros_v, wb_v, deg_sh):
    c = lax.axis_index("c")
    s = lax.axis_index("s")
    base = s * (_ZC * _CH)

    def fill(i, _):
        ones_v[i, pl.ds(0, _DW)] = jnp.full((_DW,), 1.0, jnp.float32)
        zeros_v[i, pl.ds(0, _DW)] = jnp.zeros((_DW,), jnp.float32)
        return 0
    lax.fori_loop(0, _CH, fill, 0)

    def zchunk(i, _):
        pltpu.sync_copy(zeros_v, deg_sh.at[pl.ds(base + i * _CH, _CH)])
        return 0
    lax.fori_loop(0, _ZC, zchunk, 0)

    pltpu.sync_copy(src_hbm.at[s], src_v)
    pltpu.sync_copy(dst_hbm.at[s], dst_v)
    plsc.subcore_barrier()

    @pl.when(c == 0)
    def _():
        def stepd(j, _):
            pltpu.sync_copy(ones_v, deg_sh.at[dst_v.at[j]], add=True)
            return 0
        lax.fori_loop(0, _JPW, stepd, 0)

    @pl.when(c == 1)
    def _():
        def steps_(j, _):
            pltpu.sync_copy(ones_v, deg_sh.at[src_v.at[j]], add=True)
            return 0
        lax.fori_loop(0, _JPW, steps_, 0)
    plsc.subcore_barrier()

    def wchunk(i, _):
        pltpu.sync_copy(deg_sh.at[pl.ds(base + i * _CH, _CH)], wb_v)
        pltpu.sync_copy(wb_v, out_hbm.at[c, pl.ds(base + i * _CH, _CH)])
        return 0
    lax.fori_loop(0, _ZC, wchunk, 0)


_cnt_call = functools.partial(
    pl.kernel,
    out_type=jax.ShapeDtypeStruct((_NC, _R, _DW), jnp.float32),
    mesh=_mesh,
    compiler_params=pltpu.CompilerParams(use_tc_tiling_on_sc=False),
    scratch_types=[
        pltpu.VMEM((_JPW, _CH), jnp.int32),
        pltpu.VMEM((_JPW, _CH), jnp.int32),
        pltpu.VMEM((_CH, _DW), jnp.float32),
        pltpu.VMEM((_CH, _DW), jnp.float32),
        pltpu.VMEM((_CH, _DW), jnp.float32),
        pltpu.VMEM_SHARED((_R, _DW), jnp.float32),
    ],
)(_cnt_body)


_BLK = 512


def _combine_body(a, av, bv, o):
    o[...] = a[...] * av[...] + bv[...]


def _combine_call(a, av, bv):
    spec = pl.BlockSpec((_NC, _BLK, _DH), lambda i: (0, i, 0))
    return pl.pallas_call(
        _combine_body,
        grid=(_R // _BLK,),
        in_specs=[spec] * 3,
        out_specs=spec,
        out_shape=jax.ShapeDtypeStruct((_NC, _R, _DH), jnp.float32),
    )(a, av, bv)


def _mlp_body(hv, rv, w0, b0, w1, b1, o):
    g = hv[...] * rv[...]
    h = jnp.concatenate([g[0], g[1]], axis=-1)
    z = jnp.maximum(
        jnp.dot(h, w0[...], preferred_element_type=jnp.float32) + b0[...], 0.0)
    o[...] = jnp.dot(z, w1[...], preferred_element_type=jnp.float32) + b1[...]


def _mlp_call(hv, rv, w0, b0, w1, b1):
    H = w0.shape[1]
    P = w1.shape[1]
    return pl.pallas_call(
        _mlp_body,
        grid=(_R // _BLK,),
        in_specs=[
            pl.BlockSpec((_NC, _BLK, _DH), lambda i: (0, i, 0)),
            pl.BlockSpec((_NC, _BLK, _DH), lambda i: (0, i, 0)),
            pl.BlockSpec((_D, H), lambda i: (0, 0)),
            pl.BlockSpec((1, H), lambda i: (0, 0)),
            pl.BlockSpec((H, P), lambda i: (0, 0)),
            pl.BlockSpec((1, P), lambda i: (0, 0)),
        ],
        out_specs=pl.BlockSpec((_BLK, P), lambda i: (i, 0)),
        out_shape=jax.ShapeDtypeStruct((_R, P), jnp.float32),
    )(hv, rv, w0, b0, w1, b1)


def _to_split(m):
    return m.reshape(_R, _NC, _DH).transpose(1, 0, 2)


def kernel(x, edge_index, W0, b0, W1, b1):
    src = edge_index[0].astype(jnp.int32)
    dst = edge_index[1].astype(jnp.int32)
    pad = jnp.full((_EPAD - _E,), _N, jnp.int32)
    src3 = jnp.concatenate([src, pad]).reshape(_NS, _JPW, _CH)
    dst3 = jnp.concatenate([dst, pad]).reshape(_NS, _JPW, _CH)

    dcnt = _cnt_call(src3, dst3)
    deg_in = dcnt[0, :_N, 0]
    deg_out = dcnt[1, :_N, 0]
    in_norm = lax.rsqrt(jnp.maximum(deg_in, 1.0))
    out_norm = lax.rsqrt(jnp.maximum(deg_out, 1.0))

    zpad = jnp.zeros((_R - _N,), jnp.float32)
    opad = jnp.ones((_R - _N,), jnp.float32)
    onp = jnp.concatenate([out_norm, opad])
    inp_ = jnp.concatenate([in_norm, zpad])
    xpad = jnp.pad(x, ((0, _R - _N), (0, 0)))

    scale = (1.0 - _ALPHA)
    av = _to_split(jnp.broadcast_to((scale * onp * inp_)[:, None], (_R, _D)))
    bv = _to_split((_ALPHA * onp)[:, None] * xpad)
    recip = jnp.concatenate([jnp.sqrt(jnp.maximum(deg_out, 1.0)), opad])
    rv = _to_split(jnp.broadcast_to(recip[:, None], (_R, _D)))

    def step(_, h):
        return _combine_call(_agg_call(h, src3, dst3), av, bv)

    h = lax.fori_loop(0, _K, step, _to_split(onp[:, None] * xpad))

    H = W0.shape[1]
    C = W1.shape[1]
    P = 128
    W1p = jnp.pad(W1, ((0, 0), (0, P - C)))
    b1p = jnp.pad(b1, (0, P - C)).reshape(1, P)
    b0r = b0.reshape(1, H)
    logits = _mlp_call(h, rv, W0, b0r, W1p, b1p)
    return logits[:_N, :C]
